# Initial kernel scaffold; baseline (speedup 1.0000x reference)
import jax
import jax.numpy as jnp
from jax.experimental import pallas as pl


def kernel(features, W, b):
    raise NotImplementedError("write your pallas kernel here")


# speedup vs baseline: 1.2283x; 1.2283x over previous
import functools

import jax
import jax.numpy as jnp
from jax.experimental import pallas as pl
from jax.experimental.pallas import tpu as pltpu

N, C, S = 50000, 128, 8
BN = 2000


def _route_kernel(f_ref, w_ref, b_ref, out_ref):
    f = f_ref[...]
    logits = jnp.dot(f, w_ref[...], preferred_element_type=jnp.float32)
    logits = logits + b_ref[...]
    idx = jnp.argmax(logits, axis=1)
    sel = idx[None, :, None] == jax.lax.broadcasted_iota(jnp.int32, (S, BN, 1), 0)
    out_ref[...] = jnp.where(sel, f[None, :, :], 0.0)


@functools.partial(jax.jit, static_argnames=())
def kernel(features, W, b):
    grid = (N // BN,)
    return pl.pallas_call(
        _route_kernel,
        grid=grid,
        in_specs=[
            pl.BlockSpec((BN, C), lambda i: (i, 0)),
            pl.BlockSpec((C, S), lambda i: (0, 0)),
            pl.BlockSpec((S,), lambda i: (0,)),
        ],
        out_specs=pl.BlockSpec((S, BN, C), lambda i: (0, i, 0)),
        out_shape=jax.ShapeDtypeStruct((S, N, C), jnp.float32),
        compiler_params=pltpu.CompilerParams(
            dimension_semantics=("arbitrary",),
        ),
    )(features, W, b)
